# Initial kernel scaffold; baseline (speedup 1.0000x reference)
import jax
import jax.numpy as jnp
from jax.experimental import pallas as pl


def kernel(x, edge_index, edge_attr, Wl1, Wr1, We1, att1, b1, Ws, bs, gamma, beta, Wl2, Wr2, We2, att2, b2, Wc1, bc1, Wt1, bt1, Wc2, bc2, Wt2, bt2):
    raise NotImplementedError("write your pallas kernel here")


# speedup vs baseline: 35.0679x; 35.0679x over previous
import functools

import jax
import jax.numpy as jnp
from jax import lax
from jax.experimental import pallas as pl
from jax.experimental.pallas import tpu as pltpu
from jax.experimental.pallas import tpu_sc as plsc

F32 = jnp.float32
I32 = jnp.int32

L = 16
NC = 2
NS = 16
NW = NC * NS
B = 400


def _mean_body(ea_ref, out_ref):
    i = pl.program_id(0)

    @pl.when(i == 0)
    def _():
        out_ref[...] = jnp.zeros_like(out_ref)

    out_ref[...] += jnp.sum(ea_ref[...]).reshape(1, 1)


def _ea_mean(ea, E):
    ea2 = ea.reshape(E // 128, 128)
    grid = 5
    s = pl.pallas_call(
        _mean_body,
        grid=(grid,),
        in_specs=[pl.BlockSpec((E // 128 // grid, 128), lambda i: (i, 0))],
        out_specs=pl.BlockSpec((1, 1), lambda i: (0, 0)),
        out_shape=jax.ShapeDtypeStruct((1, 1), F32),
    )(ea2)
    return s / E


def _lrelu(m):
    return jnp.maximum(m, 0.2 * m)


def _self_attn(xl, xr, mea, wev, attr):
    m = _lrelu(xl + xr + mea * wev)
    logit = jnp.sum(m * attr, axis=1, keepdims=True)
    return jnp.exp(jnp.clip(logit, -80.0, 80.0))


def _node1_body(x_ref, mea_ref, wlt_ref, wrt_ref, wev_ref, attr_ref,
                xl_ref, xr_ref, numi_ref, deni_ref):
    x = x_ref[...]
    wlt = wlt_ref[...]
    wrt = wrt_ref[...]
    xl = x[:, 0:1] * wlt[0:1, :] + x[:, 1:2] * wlt[1:2, :]
    xr = x[:, 0:1] * wrt[0:1, :] + x[:, 1:2] * wrt[1:2, :]
    ex = _self_attn(xl, xr, mea_ref[0, 0], wev_ref[...], attr_ref[...])
    xl_ref[...] = xl
    xr_ref[...] = xr
    numi_ref[...] = ex * xl
    deni_ref[...] = ex


def _node2_body(np_ref, dp_ref, numi_ref, deni_ref, x_ref, mea_ref,
                wst_ref, bias1_ref, geff_ref, beta_ref,
                wl2t_ref, wr2t_ref, wev2_ref, att2r_ref,
                xl2_ref, xr2_ref, num2i_ref, den2i_ref, exs_ref):
    num = np_ref[0] + np_ref[1] + numi_ref[...]
    den = dp_ref[0] + dp_ref[1] + deni_ref[...]
    out1 = num / (den + 1e-16)
    x = x_ref[...]
    wst = wst_ref[...]
    h = out1 + bias1_ref[...] + x[:, 0:1] * wst[0:1, :] + x[:, 1:2] * wst[1:2, :]
    h = geff_ref[...] * h + beta_ref[...]
    h = jnp.where(h > 0, h, jnp.exp(jnp.minimum(h, 0.0)) - 1.0)
    xl2 = jnp.dot(h, wl2t_ref[...], preferred_element_type=F32)
    xr2 = jnp.dot(h, wr2t_ref[...], preferred_element_type=F32)
    ex = _self_attn(xl2, xr2, mea_ref[0, 0], wev2_ref[...], att2r_ref[...])
    xl2_ref[...] = xl2
    xr2_ref[...] = xr2
    num2i_ref[...] = ex * xl2
    den2i_ref[...] = ex
    exs_ref[...] = ex


def _node3_body(np_ref, dp_ref, numi_ref, deni_ref, exs_ref, b2r_ref,
                wc1t_ref, bc1r_ref, wc2t_ref, bc2r_ref,
                wt1t_ref, bt1r_ref, wt2t_ref, bt2r_ref,
                hc_ref, ht_ref, invden_ref, aself_ref):
    num = np_ref[0] + np_ref[1] + numi_ref[...]
    den = dp_ref[0] + dp_ref[1] + deni_ref[...]
    invden = 1.0 / (den + 1e-16)
    h = num * invden + b2r_ref[...]
    h = jnp.where(h > 0, h, jnp.exp(jnp.minimum(h, 0.0)) - 1.0)
    hc = jnp.dot(h, wc1t_ref[...], preferred_element_type=F32) + bc1r_ref[...]
    hc = jnp.dot(hc, wc2t_ref[...], preferred_element_type=F32) + bc2r_ref[...]
    ht = jnp.dot(h, wt1t_ref[...], preferred_element_type=F32) + bt1r_ref[...]
    ht = jnp.dot(ht, wt2t_ref[...], preferred_element_type=F32) + bt2r_ref[...]
    hc_ref[...] = hc
    ht_ref[...] = ht
    invden_ref[...] = invden
    aself_ref[...] = exs_ref[...] * invden


def _full(shape):
    nd = len(shape)
    return pl.BlockSpec(shape, lambda i: (0,) * nd)


def _tc_node_call(body, n_out16, extra_outs, NPAD, BN, args, specs):
    grid = NPAD // BN
    outs = [jax.ShapeDtypeStruct((NPAD, 16), F32)] * n_out16
    out_specs = [pl.BlockSpec((BN, 16), lambda i: (i, 0))] * n_out16
    for w in extra_outs:
        outs.append(jax.ShapeDtypeStruct((NPAD, w), F32))
        out_specs.append(pl.BlockSpec((BN, w), lambda i: (i, 0)))
    return pl.pallas_call(
        body,
        grid=(grid,),
        in_specs=specs,
        out_specs=out_specs,
        out_shape=outs,
    )(*args)


def _edge_body(emit_ex, NPAD, STRIPE, PER_W,
               src_hbm, dst_hbm, ea_hbm, xl_hbm, xr_hbm, par_hbm,
               *refs):
    if emit_ex:
        num_out, den_out, ex_out = refs[:3]
        rest = refs[3:]
    else:
        num_out, den_out = refs[:2]
        rest = refs[2:]
    (srcv, dstv, eav, xlr, xrr, exv, parv,
     num_sh, den_sh, semL, semR) = rest

    c = lax.axis_index("c")
    s = lax.axis_index("s")
    w = s * NC + c
    row0 = s * STRIPE

    pltpu.sync_copy(par_hbm, parv)
    wev16 = parv[pl.ds(0, 16)]
    att16 = parv[pl.ds(16, 16)]

    def _zrow(i, _):
        xlr[i] = jnp.zeros((16,), F32)
        return 0

    lax.fori_loop(0, B, _zrow, 0)

    def _zden(i, _):
        eav[pl.ds(i * 16, 16)] = jnp.zeros((16,), F32)
        return 0

    lax.fori_loop(0, B // 16, _zden, 0)

    for k in range(STRIPE // B):
        pltpu.sync_copy(xlr, num_sh.at[pl.ds(row0 + k * B, B), :])
        pltpu.sync_copy(eav, den_sh.at[pl.ds(row0 + k * B, B)])
    plsc.subcore_barrier()

    ebase = w * PER_W

    def _chunk(g, _):
        e0 = ebase + g * B
        pltpu.sync_copy(src_hbm.at[pl.ds(e0, B)], srcv)
        pltpu.sync_copy(dst_hbm.at[pl.ds(e0, B)], dstv)
        pltpu.sync_copy(ea_hbm.at[pl.ds(e0, B)], eav)
        cpl = pltpu.async_copy(xl_hbm.at[srcv], xlr, semL)
        cpr = pltpu.async_copy(xr_hbm.at[dstv], xrr, semR)
        cpl.wait()
        cpr.wait()

        def _group(t, _):
            b0 = t * 16
            ea16 = eav[pl.ds(b0, 16)]
            lane = lax.iota(I32, 16)
            logitv = jnp.zeros((16,), F32)
            vls = []
            for j in range(16):
                vl = xlr[b0 + j]
                vr = xrr[b0 + j]
                v = vl + vr + ea16[j] * wev16
                v = jnp.maximum(v, 0.2 * v)
                lg = jnp.sum(v * att16)
                logitv = jnp.where(lane == j, lg, logitv)
                vls.append(vl)
            ex16 = jnp.exp(jnp.clip(logitv, -80.0, 80.0))
            exv[pl.ds(b0, 16)] = ex16
            for j in range(16):
                xlr[b0 + j] = ex16[j] * vls[j]
            return 0

        lax.fori_loop(0, B // 16, _group, 0)

        pltpu.sync_copy(xlr, num_sh.at[dstv], add=True)
        pltpu.sync_copy(exv, den_sh.at[dstv], add=True)
        if emit_ex:
            pltpu.sync_copy(exv, ex_out.at[pl.ds(e0, B)])
        return 0

    lax.fori_loop(0, PER_W // B, _chunk, 0)
    plsc.subcore_barrier()

    for k in range(STRIPE // B):
        r0 = row0 + k * B
        pltpu.sync_copy(num_sh.at[pl.ds(r0, B), :],
                        num_out.at[c, pl.ds(r0, B), :])
        pltpu.sync_copy(den_sh.at[pl.ds(r0, B)],
                        den_out.at[c, pl.ds(r0, B)])


def _edge_pass(src, dst, ea, xl, xr, params, E, NPAD, STRIPE, emit_ex):
    PER_W = E // NW
    mesh = plsc.VectorSubcoreMesh(core_axis_name="c", subcore_axis_name="s")
    out_type = [jax.ShapeDtypeStruct((NC, NPAD, 16), F32),
                jax.ShapeDtypeStruct((NC, NPAD), F32)]
    if emit_ex:
        out_type.append(jax.ShapeDtypeStruct((E,), F32))
    scratch = [
        pltpu.VMEM((B,), I32),
        pltpu.VMEM((B,), I32),
        pltpu.VMEM((B,), F32),
        pltpu.VMEM((B, 16), F32),
        pltpu.VMEM((B, 16), F32),
        pltpu.VMEM((B,), F32),
        pltpu.VMEM((32,), F32),
        pltpu.VMEM_SHARED((NPAD, 16), F32),
        pltpu.VMEM_SHARED((NPAD,), F32),
        pltpu.SemaphoreType.DMA,
        pltpu.SemaphoreType.DMA,
    ]
    body = functools.partial(_edge_body, emit_ex, NPAD, STRIPE, PER_W)
    fn = pl.kernel(body, out_type=tuple(out_type), mesh=mesh,
                   scratch_types=tuple(scratch),
                   compiler_params=pltpu.CompilerParams(
                       needs_layout_passes=False, use_tc_tiling_on_sc=False))
    return fn(src, dst, ea, xl, xr, params)


def _alpha_body(PER_W, dst_hbm, ex_hbm, invden_hbm, alpha_out,
                dstv, exv, dvals, av, sem):
    c = lax.axis_index("c")
    s = lax.axis_index("s")
    w = s * NC + c
    ebase = w * PER_W

    def _chunk(g, _):
        e0 = ebase + g * B
        pltpu.sync_copy(dst_hbm.at[pl.ds(e0, B)], dstv)
        pltpu.sync_copy(ex_hbm.at[pl.ds(e0, B)], exv)
        pltpu.async_copy(invden_hbm.at[dstv], dvals, sem).wait()

        def _group(t, _):
            b0 = t * 16
            av[pl.ds(b0, 16)] = exv[pl.ds(b0, 16)] * dvals[pl.ds(b0, 16)]
            return 0

        lax.fori_loop(0, B // 16, _group, 0)
        pltpu.sync_copy(av, alpha_out.at[pl.ds(e0, B)])
        return 0

    lax.fori_loop(0, PER_W // B, _chunk, 0)


def _alpha_pass(dst, ex, invden, E):
    PER_W = E // NW
    mesh = plsc.VectorSubcoreMesh(core_axis_name="c", subcore_axis_name="s")
    scratch = [
        pltpu.VMEM((B,), I32),
        pltpu.VMEM((B,), F32),
        pltpu.VMEM((B,), F32),
        pltpu.VMEM((B,), F32),
        pltpu.SemaphoreType.DMA,
    ]
    fn = pl.kernel(functools.partial(_alpha_body, PER_W),
                   out_type=jax.ShapeDtypeStruct((E,), F32),
                   mesh=mesh, scratch_types=tuple(scratch),
                   compiler_params=pltpu.CompilerParams(
                       needs_layout_passes=False, use_tc_tiling_on_sc=False))
    return fn(dst, ex, invden)


def kernel(x, edge_index, edge_attr, Wl1, Wr1, We1, att1, b1, Ws, bs,
           gamma, beta, Wl2, Wr2, We2, att2, b2, Wc1, bc1, Wt1, bt1,
           Wc2, bc2, Wt2, bt2):
    N = x.shape[0]
    E = edge_index.shape[1]
    STRIPE = (((N + NS - 1) // NS) + B - 1) // B * B
    NPAD = NS * STRIPE
    BN = 512
    assert E % (NW * B) == 0 and NPAD % BN == 0

    src = edge_index[0]
    dst = edge_index[1]
    ea = edge_attr[:, 0]
    xpad = jnp.pad(x, ((0, NPAD - N), (0, 0)))

    mea = _ea_mean(ea, E)

    xl1, xr1, numi1, deni1 = _tc_node_call(
        _node1_body, 2, [16, 1], NPAD, BN,
        (xpad, mea, Wl1.T, Wr1.T, We1.reshape(1, 16), att1.reshape(1, 16)),
        [pl.BlockSpec((BN, 2), lambda i: (i, 0)), _full((1, 1)),
         _full((2, 16)), _full((2, 16)), _full((1, 16)), _full((1, 16))],
    )

    params1 = jnp.concatenate([We1[:, 0], att1]).astype(F32)
    np1, dp1 = _edge_pass(src, dst, ea, xl1, xr1, params1,
                          E, NPAD, STRIPE, emit_ex=False)

    bias1 = (b1 + bs).reshape(1, 16)
    geff = (gamma / jnp.sqrt(1.0 + 1e-5)).reshape(1, 16)
    xl2, xr2, num2i, den2i, exs = _tc_node_call(
        _node2_body, 2, [16, 1, 1], NPAD, BN,
        (np1, dp1.reshape(NC, NPAD, 1), numi1, deni1, xpad, mea,
         Ws.T, bias1, geff, beta.reshape(1, 16),
         Wl2.T, Wr2.T, We2.reshape(1, 16), att2.reshape(1, 16)),
        [pl.BlockSpec((NC, BN, 16), lambda i: (0, i, 0)),
         pl.BlockSpec((NC, BN, 1), lambda i: (0, i, 0)),
         pl.BlockSpec((BN, 16), lambda i: (i, 0)),
         pl.BlockSpec((BN, 1), lambda i: (i, 0)),
         pl.BlockSpec((BN, 2), lambda i: (i, 0)), _full((1, 1)),
         _full((2, 16)), _full((1, 16)), _full((1, 16)), _full((1, 16)),
         _full((16, 16)), _full((16, 16)), _full((1, 16)), _full((1, 16))],
    )

    params2 = jnp.concatenate([We2[:, 0], att2]).astype(F32)
    np2, dp2, ex2 = _edge_pass(src, dst, ea, xl2, xr2, params2,
                               E, NPAD, STRIPE, emit_ex=True)

    hc, ht, invden, aself = _tc_node_call(
        _node3_body, 0, [9, 4, 1, 1], NPAD, BN,
        (np2, dp2.reshape(NC, NPAD, 1), num2i, den2i, exs,
         b2.reshape(1, 16),
         Wc1.T, bc1.reshape(1, 16), Wc2.T, bc2.reshape(1, 9),
         Wt1.T, bt1.reshape(1, 16), Wt2.T, bt2.reshape(1, 4)),
        [pl.BlockSpec((NC, BN, 16), lambda i: (0, i, 0)),
         pl.BlockSpec((NC, BN, 1), lambda i: (0, i, 0)),
         pl.BlockSpec((BN, 16), lambda i: (i, 0)),
         pl.BlockSpec((BN, 1), lambda i: (i, 0)),
         pl.BlockSpec((BN, 1), lambda i: (i, 0)),
         _full((1, 16)),
         _full((16, 16)), _full((1, 16)), _full((16, 9)), _full((1, 9)),
         _full((16, 16)), _full((1, 16)), _full((16, 4)), _full((1, 4))],
    )

    alpha_e = _alpha_pass(dst, ex2, invden[:, 0], E)

    out13 = jnp.concatenate([hc[:N], ht[:N]], axis=1)
    alpha = jnp.concatenate([alpha_e, aself[:N, 0]])
    return (out13, alpha)


# speedup vs baseline: 46.8277x; 1.3353x over previous
import functools

import jax
import jax.numpy as jnp
from jax import lax
from jax.experimental import pallas as pl
from jax.experimental.pallas import tpu as pltpu
from jax.experimental.pallas import tpu_sc as plsc

F32 = jnp.float32
I32 = jnp.int32

L = 16
NC = 2
NS = 16
NW = NC * NS
B = 304


def _mean_body(ea_ref, out_ref):
    i = pl.program_id(0)

    @pl.when(i == 0)
    def _():
        out_ref[...] = jnp.zeros_like(out_ref)

    out_ref[...] += jnp.sum(ea_ref[...]).reshape(1, 1)


def _ea_mean(ea, E):
    rows = ea.shape[0] // 128
    ea2 = ea.reshape(rows, 128)
    grid = next(g for g in (8, 5, 4, 2, 1)
                if rows % g == 0 and (rows // g) % 8 == 0)
    s = pl.pallas_call(
        _mean_body,
        grid=(grid,),
        in_specs=[pl.BlockSpec((rows // grid, 128), lambda i: (i, 0))],
        out_specs=pl.BlockSpec((1, 1), lambda i: (0, 0)),
        out_shape=jax.ShapeDtypeStruct((1, 1), F32),
    )(ea2)
    return s / E


def _lrelu(m):
    return jnp.maximum(m, 0.2 * m)


def _self_attn(xl, xr, mea, wev, attr):
    m = _lrelu(xl + xr + mea * wev)
    logit = jnp.sum(m * attr, axis=1, keepdims=True)
    return jnp.exp(jnp.clip(logit, -80.0, 80.0))


def _node1_body(x_ref, mea_ref, wlt_ref, wrt_ref, wev_ref, attr_ref,
                xl_ref, xr_ref, numi_ref, deni_ref):
    x = x_ref[...]
    wlt = wlt_ref[...]
    wrt = wrt_ref[...]
    xl = x[:, 0:1] * wlt[0:1, :] + x[:, 1:2] * wlt[1:2, :]
    xr = x[:, 0:1] * wrt[0:1, :] + x[:, 1:2] * wrt[1:2, :]
    ex = _self_attn(xl, xr, mea_ref[0, 0], wev_ref[...], attr_ref[...])
    xl_ref[...] = xl
    xr_ref[...] = xr
    numi_ref[...] = ex * xl
    deni_ref[...] = ex


def _node2_body(np_ref, dp_ref, numi_ref, deni_ref, x_ref, mea_ref,
                wst_ref, bias1_ref, geff_ref, beta_ref,
                wl2t_ref, wr2t_ref, wev2_ref, att2r_ref,
                xl2_ref, xr2_ref, num2i_ref, den2i_ref, exs_ref):
    num = np_ref[0] + np_ref[1] + numi_ref[...]
    den = dp_ref[0] + dp_ref[1] + deni_ref[...]
    out1 = num / (den + 1e-16)
    x = x_ref[...]
    wst = wst_ref[...]
    h = out1 + bias1_ref[...] + x[:, 0:1] * wst[0:1, :] + x[:, 1:2] * wst[1:2, :]
    h = geff_ref[...] * h + beta_ref[...]
    h = jnp.where(h > 0, h, jnp.exp(jnp.minimum(h, 0.0)) - 1.0)
    xl2 = jnp.dot(h, wl2t_ref[...], preferred_element_type=F32)
    xr2 = jnp.dot(h, wr2t_ref[...], preferred_element_type=F32)
    ex = _self_attn(xl2, xr2, mea_ref[0, 0], wev2_ref[...], att2r_ref[...])
    xl2_ref[...] = xl2
    xr2_ref[...] = xr2
    num2i_ref[...] = ex * xl2
    den2i_ref[...] = ex
    exs_ref[...] = ex


def _node3_body(np_ref, dp_ref, numi_ref, deni_ref, exs_ref, b2r_ref,
                wc1t_ref, bc1r_ref, wc2t_ref, bc2r_ref,
                wt1t_ref, bt1r_ref, wt2t_ref, bt2r_ref,
                hc_ref, ht_ref, invden_ref, aself_ref):
    num = np_ref[0] + np_ref[1] + numi_ref[...]
    den = dp_ref[0] + dp_ref[1] + deni_ref[...]
    invden = 1.0 / (den + 1e-16)
    h = num * invden + b2r_ref[...]
    h = jnp.where(h > 0, h, jnp.exp(jnp.minimum(h, 0.0)) - 1.0)
    hc = jnp.dot(h, wc1t_ref[...], preferred_element_type=F32) + bc1r_ref[...]
    hc = jnp.dot(hc, wc2t_ref[...], preferred_element_type=F32) + bc2r_ref[...]
    ht = jnp.dot(h, wt1t_ref[...], preferred_element_type=F32) + bt1r_ref[...]
    ht = jnp.dot(ht, wt2t_ref[...], preferred_element_type=F32) + bt2r_ref[...]
    hc_ref[...] = hc
    ht_ref[...] = ht
    invden_ref[...] = invden
    aself_ref[...] = exs_ref[...] * invden


def _full(shape):
    nd = len(shape)
    return pl.BlockSpec(shape, lambda i: (0,) * nd)


def _tc_node_call(body, n_out16, extra_outs, NPAD, BN, args, specs):
    grid = NPAD // BN
    outs = [jax.ShapeDtypeStruct((NPAD, 16), F32)] * n_out16
    out_specs = [pl.BlockSpec((BN, 16), lambda i: (i, 0))] * n_out16
    for w in extra_outs:
        outs.append(jax.ShapeDtypeStruct((NPAD, w), F32))
        out_specs.append(pl.BlockSpec((BN, w), lambda i: (i, 0)))
    return pl.pallas_call(
        body,
        grid=(grid,),
        in_specs=specs,
        out_specs=out_specs,
        out_shape=outs,
    )(*args)


def _edge_body(emit_ex, NPAD, STRIPE, PER_W,
               src_hbm, dst_hbm, ea_hbm, xl_hbm, xr_hbm, par_hbm,
               *refs):
    if emit_ex:
        num_out, den_out, ex_out = refs[:3]
        rest = refs[3:]
    else:
        num_out, den_out = refs[:2]
        rest = refs[2:]
    (srcv0, srcv1, dstv0, dstv1, eav0, eav1, xlr0, xlr1, xrr0, xrr1,
     exv0, exv1, parv, num_sh, den_sh, semI0, semI1, semL0, semL1,
     semR0, semR1) = rest
    srcv = (srcv0, srcv1)
    dstv = (dstv0, dstv1)
    eav = (eav0, eav1)
    xlr = (xlr0, xlr1)
    xrr = (xrr0, xrr1)
    exv = (exv0, exv1)
    semI = (semI0, semI1)
    semL = (semL0, semL1)
    semR = (semR0, semR1)

    c = lax.axis_index("c")
    s = lax.axis_index("s")
    w = s * NC + c
    row0 = s * STRIPE

    pltpu.sync_copy(par_hbm, parv)
    wev16 = parv[pl.ds(0, 16)]
    att16 = parv[pl.ds(16, 16)]

    def _zrow(i, _):
        xlr0[i] = jnp.zeros((16,), F32)
        return 0

    lax.fori_loop(0, B, _zrow, 0)

    def _zden(i, _):
        eav0[pl.ds(i * 16, 16)] = jnp.zeros((16,), F32)
        return 0

    lax.fori_loop(0, B // 16, _zden, 0)

    off = 0
    while off < STRIPE:
        sz = min(B, STRIPE - off)
        pltpu.sync_copy(xlr0.at[pl.ds(0, sz), :],
                        num_sh.at[pl.ds(row0 + off, sz), :])
        pltpu.sync_copy(eav0.at[pl.ds(0, sz)],
                        den_sh.at[pl.ds(row0 + off, sz)])
        off += sz
    plsc.subcore_barrier()

    ebase = w * PER_W
    NCH = PER_W // B

    def _start_lin(g, b):
        e0 = ebase + g * B
        pltpu.async_copy(src_hbm.at[pl.ds(e0, B)], srcv[b], semI[b])
        pltpu.async_copy(dst_hbm.at[pl.ds(e0, B)], dstv[b], semI[b])
        pltpu.async_copy(ea_hbm.at[pl.ds(e0, B)], eav[b], semI[b])

    def _wait_lin(b):
        pltpu.make_async_copy(src_hbm.at[pl.ds(0, B)], srcv[b], semI[b]).wait()
        pltpu.make_async_copy(dst_hbm.at[pl.ds(0, B)], dstv[b], semI[b]).wait()
        pltpu.make_async_copy(ea_hbm.at[pl.ds(0, B)], eav[b], semI[b]).wait()

    def _start_gather(b):
        pltpu.async_copy(xl_hbm.at[srcv[b]], xlr[b], semL[b])
        pltpu.async_copy(xr_hbm.at[dstv[b]], xrr[b], semR[b])

    def _wait_gather(b):
        pltpu.make_async_copy(xl_hbm.at[srcv[b]], xlr[b], semL[b]).wait()
        pltpu.make_async_copy(xr_hbm.at[dstv[b]], xrr[b], semR[b]).wait()

    def _compute(b):
        xlrb = xlr[b]
        xrrb = xrr[b]
        eavb = eav[b]
        exvb = exv[b]

        def _group(t, _):
            b0 = t * 16
            ea16 = eavb[pl.ds(b0, 16)]
            lane = lax.iota(I32, 16)
            logitv = jnp.zeros((16,), F32)
            vls = []
            for j in range(16):
                vl = xlrb[b0 + j]
                vr = xrrb[b0 + j]
                v = vl + vr + ea16[j] * wev16
                v = jnp.maximum(v, 0.2 * v)
                lg = jnp.sum(v * att16)
                logitv = jnp.where(lane == j, lg, logitv)
                vls.append(vl)
            ex16 = jnp.exp(jnp.clip(logitv, -80.0, 80.0))
            exvb[pl.ds(b0, 16)] = ex16
            for j in range(16):
                xlrb[b0 + j] = ex16[j] * vls[j]
            return 0

        lax.fori_loop(0, B // 16, _group, 0)

    def _scatter(g, b):
        pltpu.sync_copy(xlr[b], num_sh.at[dstv[b]], add=True)
        pltpu.sync_copy(exv[b], den_sh.at[dstv[b]], add=True)
        if emit_ex:
            e0 = ebase + g * B
            pltpu.sync_copy(exv[b], ex_out.at[pl.ds(e0, B)])

    _start_lin(0, 0)
    _wait_lin(0)
    _start_gather(0)
    _start_lin(1, 1)

    def _iter2b(g2, _):
        for b in (0, 1):
            g = g2 * 2 + b
            nb = 1 - b
            _wait_gather(b)

            @pl.when(g + 1 < NCH)
            def _wg():
                _wait_lin(nb)
                _start_gather(nb)

            _compute(b)
            _scatter(g, b)

            @pl.when(g + 2 < NCH)
            def _sl():
                _start_lin(g + 2, b)
        return 0

    lax.fori_loop(0, NCH // 2, _iter2b, 0)
    plsc.subcore_barrier()

    pltpu.sync_copy(num_sh.at[pl.ds(row0, STRIPE), :],
                    num_out.at[c, pl.ds(row0, STRIPE), :])
    pltpu.sync_copy(den_sh.at[pl.ds(row0, STRIPE)],
                    den_out.at[c, pl.ds(row0, STRIPE)])


def _edge_pass(src, dst, ea, xl, xr, params, E, NPAD, STRIPE, emit_ex):
    PER_W = E // NW
    mesh = plsc.VectorSubcoreMesh(core_axis_name="c", subcore_axis_name="s")
    out_type = [jax.ShapeDtypeStruct((NC, NPAD, 16), F32),
                jax.ShapeDtypeStruct((NC, NPAD), F32)]
    if emit_ex:
        out_type.append(jax.ShapeDtypeStruct((E,), F32))
    scratch = (
        [pltpu.VMEM((B,), I32)] * 4 +
        [pltpu.VMEM((B,), F32)] * 2 +
        [pltpu.VMEM((B, 16), F32)] * 4 +
        [pltpu.VMEM((B,), F32)] * 2 +
        [pltpu.VMEM((32,), F32)] +
        [pltpu.VMEM_SHARED((NPAD, 16), F32),
         pltpu.VMEM_SHARED((NPAD,), F32)] +
        [pltpu.SemaphoreType.DMA] * 6
    )
    body = functools.partial(_edge_body, emit_ex, NPAD, STRIPE, PER_W)
    fn = pl.kernel(body, out_type=tuple(out_type), mesh=mesh,
                   scratch_types=tuple(scratch),
                   compiler_params=pltpu.CompilerParams(
                       needs_layout_passes=False, use_tc_tiling_on_sc=False))
    return fn(src, dst, ea, xl, xr, params)


def _alpha_body(PER_W, dst_hbm, ex_hbm, invden_hbm, alpha_out,
                dstv0, dstv1, exv0, exv1, dvals0, dvals1, av0, av1,
                semI0, semI1, semG0, semG1):
    dstv = (dstv0, dstv1)
    exv = (exv0, exv1)
    dvals = (dvals0, dvals1)
    av = (av0, av1)
    semI = (semI0, semI1)
    semG = (semG0, semG1)
    c = lax.axis_index("c")
    s = lax.axis_index("s")
    w = s * NC + c
    ebase = w * PER_W
    NCH = PER_W // B

    def _start_lin(g, b):
        e0 = ebase + g * B
        pltpu.async_copy(dst_hbm.at[pl.ds(e0, B)], dstv[b], semI[b])
        pltpu.async_copy(ex_hbm.at[pl.ds(e0, B)], exv[b], semI[b])

    def _wait_lin(b):
        pltpu.make_async_copy(dst_hbm.at[pl.ds(0, B)], dstv[b], semI[b]).wait()
        pltpu.make_async_copy(ex_hbm.at[pl.ds(0, B)], exv[b], semI[b]).wait()

    def _start_gather(b):
        pltpu.async_copy(invden_hbm.at[dstv[b]], dvals[b], semG[b])

    def _wait_gather(b):
        pltpu.make_async_copy(invden_hbm.at[dstv[b]], dvals[b],
                              semG[b]).wait()

    def _compute_out(g, b):
        exvb = exv[b]
        dvalsb = dvals[b]
        avb = av[b]

        def _group(t, _):
            b0 = t * 16
            avb[pl.ds(b0, 16)] = exvb[pl.ds(b0, 16)] * dvalsb[pl.ds(b0, 16)]
            return 0

        lax.fori_loop(0, B // 16, _group, 0)
        e0 = ebase + g * B
        pltpu.sync_copy(avb, alpha_out.at[pl.ds(e0, B)])

    _start_lin(0, 0)
    _wait_lin(0)
    _start_gather(0)
    _start_lin(1, 1)

    def _iter2(g2, _):
        for b in (0, 1):
            g = g2 * 2 + b
            nb = 1 - b
            _wait_gather(b)

            @pl.when(g + 1 < NCH)
            def _wg():
                _wait_lin(nb)
                _start_gather(nb)

            _compute_out(g, b)

            @pl.when(g + 2 < NCH)
            def _sl():
                _start_lin(g + 2, b)
        return 0

    lax.fori_loop(0, NCH // 2, _iter2, 0)


def _alpha_pass(dst, ex, invden, E):
    PER_W = E // NW
    mesh = plsc.VectorSubcoreMesh(core_axis_name="c", subcore_axis_name="s")
    scratch = (
        [pltpu.VMEM((B,), I32)] * 2 +
        [pltpu.VMEM((B,), F32)] * 6 +
        [pltpu.SemaphoreType.DMA] * 4
    )
    fn = pl.kernel(functools.partial(_alpha_body, PER_W),
                   out_type=jax.ShapeDtypeStruct((E,), F32),
                   mesh=mesh, scratch_types=tuple(scratch),
                   compiler_params=pltpu.CompilerParams(
                       needs_layout_passes=False, use_tc_tiling_on_sc=False))
    return fn(dst, ex, invden)


def kernel(x, edge_index, edge_attr, Wl1, Wr1, We1, att1, b1, Ws, bs,
           gamma, beta, Wl2, Wr2, We2, att2, b2, Wc1, bc1, Wt1, bt1,
           Wc2, bc2, Wt2, bt2):
    N = x.shape[0]
    E = edge_index.shape[1]
    STRIPE = (((N + NS - 1) // NS) + 399) // 400 * 400
    NPAD = NS * STRIPE
    BN = 512
    assert NPAD % BN == 0

    nch = (E + NW * B - 1) // (NW * B)
    nch += nch % 2
    E_PAD = NW * B * nch

    src = jnp.pad(edge_index[0], (0, E_PAD - E), constant_values=N)
    dst = jnp.pad(edge_index[1], (0, E_PAD - E), constant_values=N)
    ea = jnp.pad(edge_attr[:, 0], (0, E_PAD - E))
    xpad = jnp.pad(x, ((0, NPAD - N), (0, 0)))

    mea = _ea_mean(ea, E)

    xl1, xr1, numi1, deni1 = _tc_node_call(
        _node1_body, 2, [16, 1], NPAD, BN,
        (xpad, mea, Wl1.T, Wr1.T, We1.reshape(1, 16), att1.reshape(1, 16)),
        [pl.BlockSpec((BN, 2), lambda i: (i, 0)), _full((1, 1)),
         _full((2, 16)), _full((2, 16)), _full((1, 16)), _full((1, 16))],
    )

    params1 = jnp.concatenate([We1[:, 0], att1]).astype(F32)
    np1, dp1 = _edge_pass(src, dst, ea, xl1, xr1, params1,
                          E_PAD, NPAD, STRIPE, emit_ex=False)

    bias1 = (b1 + bs).reshape(1, 16)
    geff = (gamma / jnp.sqrt(1.0 + 1e-5)).reshape(1, 16)
    xl2, xr2, num2i, den2i, exs = _tc_node_call(
        _node2_body, 2, [16, 1, 1], NPAD, BN,
        (np1, dp1.reshape(NC, NPAD, 1), numi1, deni1, xpad, mea,
         Ws.T, bias1, geff, beta.reshape(1, 16),
         Wl2.T, Wr2.T, We2.reshape(1, 16), att2.reshape(1, 16)),
        [pl.BlockSpec((NC, BN, 16), lambda i: (0, i, 0)),
         pl.BlockSpec((NC, BN, 1), lambda i: (0, i, 0)),
         pl.BlockSpec((BN, 16), lambda i: (i, 0)),
         pl.BlockSpec((BN, 1), lambda i: (i, 0)),
         pl.BlockSpec((BN, 2), lambda i: (i, 0)), _full((1, 1)),
         _full((2, 16)), _full((1, 16)), _full((1, 16)), _full((1, 16)),
         _full((16, 16)), _full((16, 16)), _full((1, 16)), _full((1, 16))],
    )

    params2 = jnp.concatenate([We2[:, 0], att2]).astype(F32)
    np2, dp2, ex2 = _edge_pass(src, dst, ea, xl2, xr2, params2,
                               E_PAD, NPAD, STRIPE, emit_ex=True)

    hc, ht, invden, aself = _tc_node_call(
        _node3_body, 0, [9, 4, 1, 1], NPAD, BN,
        (np2, dp2.reshape(NC, NPAD, 1), num2i, den2i, exs,
         b2.reshape(1, 16),
         Wc1.T, bc1.reshape(1, 16), Wc2.T, bc2.reshape(1, 9),
         Wt1.T, bt1.reshape(1, 16), Wt2.T, bt2.reshape(1, 4)),
        [pl.BlockSpec((NC, BN, 16), lambda i: (0, i, 0)),
         pl.BlockSpec((NC, BN, 1), lambda i: (0, i, 0)),
         pl.BlockSpec((BN, 16), lambda i: (i, 0)),
         pl.BlockSpec((BN, 1), lambda i: (i, 0)),
         pl.BlockSpec((BN, 1), lambda i: (i, 0)),
         _full((1, 16)),
         _full((16, 16)), _full((1, 16)), _full((16, 9)), _full((1, 9)),
         _full((16, 16)), _full((1, 16)), _full((16, 4)), _full((1, 4))],
    )

    alpha_e = _alpha_pass(dst, ex2, invden[:, 0], E_PAD)

    out13 = jnp.concatenate([hc[:N], ht[:N]], axis=1)
    alpha = jnp.concatenate([alpha_e[:E], aself[:N, 0]])
    return (out13, alpha)


# speedup vs baseline: 49.7375x; 1.0621x over previous
import functools

import jax
import jax.numpy as jnp
from jax import lax
from jax.experimental import pallas as pl
from jax.experimental.pallas import tpu as pltpu
from jax.experimental.pallas import tpu_sc as plsc

F32 = jnp.float32
I32 = jnp.int32

L = 16
NC = 2
NS = 16
NW = NC * NS
B = 304


def _mean_body(ea_ref, out_ref):
    i = pl.program_id(0)

    @pl.when(i == 0)
    def _():
        out_ref[...] = jnp.zeros_like(out_ref)

    out_ref[...] += jnp.sum(ea_ref[...]).reshape(1, 1)


def _ea_mean(ea, E):
    rows = ea.shape[0] // 128
    ea2 = ea.reshape(rows, 128)
    grid = next(g for g in (8, 5, 4, 2, 1)
                if rows % g == 0 and (rows // g) % 8 == 0)
    s = pl.pallas_call(
        _mean_body,
        grid=(grid,),
        in_specs=[pl.BlockSpec((rows // grid, 128), lambda i: (i, 0))],
        out_specs=pl.BlockSpec((1, 1), lambda i: (0, 0)),
        out_shape=jax.ShapeDtypeStruct((1, 1), F32),
    )(ea2)
    return s / E


def _lrelu(m):
    return jnp.maximum(m, 0.2 * m)


def _self_attn(xl, xr, mea, wev, attr):
    m = _lrelu(xl + xr + mea * wev)
    logit = jnp.sum(m * attr, axis=1, keepdims=True)
    return jnp.exp(jnp.clip(logit, -80.0, 80.0))


def _node1_body(x_ref, mea_ref, wlt_ref, wrt_ref, wev_ref, attr_ref,
                xl_ref, xr_ref, numi_ref, deni_ref):
    x = x_ref[...]
    wlt = wlt_ref[...]
    wrt = wrt_ref[...]
    xl = x[:, 0:1] * wlt[0:1, :] + x[:, 1:2] * wlt[1:2, :]
    xr = x[:, 0:1] * wrt[0:1, :] + x[:, 1:2] * wrt[1:2, :]
    ex = _self_attn(xl, xr, mea_ref[0, 0], wev_ref[...], attr_ref[...])
    xl_ref[...] = xl
    xr_ref[...] = xr
    numi_ref[...] = ex * xl
    deni_ref[...] = ex


def _node2_body(np_ref, dp_ref, numi_ref, deni_ref, x_ref, mea_ref,
                wst_ref, bias1_ref, geff_ref, beta_ref,
                wl2t_ref, wr2t_ref, wev2_ref, att2r_ref,
                xl2_ref, xr2_ref, num2i_ref, den2i_ref, exs_ref):
    num = np_ref[0] + np_ref[1] + numi_ref[...]
    den = dp_ref[0] + dp_ref[1] + deni_ref[...]
    out1 = num / (den + 1e-16)
    x = x_ref[...]
    wst = wst_ref[...]
    h = out1 + bias1_ref[...] + x[:, 0:1] * wst[0:1, :] + x[:, 1:2] * wst[1:2, :]
    h = geff_ref[...] * h + beta_ref[...]
    h = jnp.where(h > 0, h, jnp.exp(jnp.minimum(h, 0.0)) - 1.0)
    xl2 = jnp.dot(h, wl2t_ref[...], preferred_element_type=F32)
    xr2 = jnp.dot(h, wr2t_ref[...], preferred_element_type=F32)
    ex = _self_attn(xl2, xr2, mea_ref[0, 0], wev2_ref[...], att2r_ref[...])
    xl2_ref[...] = xl2
    xr2_ref[...] = xr2
    num2i_ref[...] = ex * xl2
    den2i_ref[...] = ex
    exs_ref[...] = ex


def _node3_body(np_ref, dp_ref, numi_ref, deni_ref, exs_ref, b2r_ref,
                wc1t_ref, bc1r_ref, wc2t_ref, bc2r_ref,
                wt1t_ref, bt1r_ref, wt2t_ref, bt2r_ref,
                hc_ref, ht_ref, invden_ref, aself_ref):
    num = np_ref[0] + np_ref[1] + numi_ref[...]
    den = dp_ref[0] + dp_ref[1] + deni_ref[...]
    invden = 1.0 / (den + 1e-16)
    h = num * invden + b2r_ref[...]
    h = jnp.where(h > 0, h, jnp.exp(jnp.minimum(h, 0.0)) - 1.0)
    hc = jnp.dot(h, wc1t_ref[...], preferred_element_type=F32) + bc1r_ref[...]
    hc = jnp.dot(hc, wc2t_ref[...], preferred_element_type=F32) + bc2r_ref[...]
    ht = jnp.dot(h, wt1t_ref[...], preferred_element_type=F32) + bt1r_ref[...]
    ht = jnp.dot(ht, wt2t_ref[...], preferred_element_type=F32) + bt2r_ref[...]
    hc_ref[...] = hc
    ht_ref[...] = ht
    invden_ref[...] = invden
    aself_ref[...] = exs_ref[...] * invden


def _full(shape):
    nd = len(shape)
    return pl.BlockSpec(shape, lambda i: (0,) * nd)


def _tc_node_call(body, n_out16, extra_outs, NPAD, BN, args, specs):
    grid = NPAD // BN
    outs = [jax.ShapeDtypeStruct((NPAD, 16), F32)] * n_out16
    out_specs = [pl.BlockSpec((BN, 16), lambda i: (i, 0))] * n_out16
    for w in extra_outs:
        outs.append(jax.ShapeDtypeStruct((NPAD, w), F32))
        out_specs.append(pl.BlockSpec((BN, w), lambda i: (i, 0)))
    return pl.pallas_call(
        body,
        grid=(grid,),
        in_specs=specs,
        out_specs=out_specs,
        out_shape=outs,
    )(*args)


def _edge_body(emit_ex, NPAD, STRIPE, PER_W,
               src_hbm, dst_hbm, ea_hbm, xl_hbm, xr_hbm, par_hbm,
               *refs):
    if emit_ex:
        num_out, den_out, ex_out = refs[:3]
        rest = refs[3:]
    else:
        num_out, den_out = refs[:2]
        rest = refs[2:]
    (srcv0, srcv1, dstv0, dstv1, eav0, eav1, xlr0, xlr1, xrr0, xrr1,
     exv0, exv1, parv, num_sh, den_sh, semI0, semI1, semL0, semL1,
     semR0, semR1) = rest
    srcv = (srcv0, srcv1)
    dstv = (dstv0, dstv1)
    eav = (eav0, eav1)
    xlr = (xlr0, xlr1)
    xrr = (xrr0, xrr1)
    exv = (exv0, exv1)
    semI = (semI0, semI1)
    semL = (semL0, semL1)
    semR = (semR0, semR1)

    c = lax.axis_index("c")
    s = lax.axis_index("s")
    w = s * NC + c
    row0 = s * STRIPE

    pltpu.sync_copy(par_hbm, parv)
    wev16 = parv[pl.ds(0, 16)]
    att16 = parv[pl.ds(16, 16)]

    def _zrow(i, _):
        xlr0[i] = jnp.zeros((16,), F32)
        return 0

    lax.fori_loop(0, B, _zrow, 0)

    def _zden(i, _):
        eav0[pl.ds(i * 16, 16)] = jnp.zeros((16,), F32)
        return 0

    lax.fori_loop(0, B // 16, _zden, 0)

    off = 0
    while off < STRIPE:
        sz = min(B, STRIPE - off)
        pltpu.sync_copy(xlr0.at[pl.ds(0, sz), :],
                        num_sh.at[pl.ds(row0 + off, sz), :])
        pltpu.sync_copy(eav0.at[pl.ds(0, sz)],
                        den_sh.at[pl.ds(row0 + off, sz)])
        off += sz
    plsc.subcore_barrier()

    ebase = w * PER_W
    NCH = PER_W // B

    def _start_lin(g, b):
        e0 = ebase + g * B
        pltpu.async_copy(src_hbm.at[pl.ds(e0, B)], srcv[b], semI[b])
        pltpu.async_copy(dst_hbm.at[pl.ds(e0, B)], dstv[b], semI[b])
        pltpu.async_copy(ea_hbm.at[pl.ds(e0, B)], eav[b], semI[b])

    def _wait_lin(b):
        pltpu.make_async_copy(src_hbm.at[pl.ds(0, B)], srcv[b], semI[b]).wait()
        pltpu.make_async_copy(dst_hbm.at[pl.ds(0, B)], dstv[b], semI[b]).wait()
        pltpu.make_async_copy(ea_hbm.at[pl.ds(0, B)], eav[b], semI[b]).wait()

    def _start_gather(b):
        pltpu.async_copy(xl_hbm.at[srcv[b]], xlr[b], semL[b])
        pltpu.async_copy(xr_hbm.at[dstv[b]], xrr[b], semR[b])

    def _wait_gather(b):
        pltpu.make_async_copy(xl_hbm.at[srcv[b]], xlr[b], semL[b]).wait()
        pltpu.make_async_copy(xr_hbm.at[dstv[b]], xrr[b], semR[b]).wait()

    def _compute(b):
        xlrb = xlr[b]
        xrrb = xrr[b]
        eavb = eav[b]
        exvb = exv[b]

        def _group(t, _):
            b0 = t * 16
            ea16 = eavb[pl.ds(b0, 16)]
            lane = lax.iota(I32, 16)
            logitv = jnp.zeros((16,), F32)
            vls = []
            for j in range(16):
                vl = xlrb[b0 + j]
                vr = xrrb[b0 + j]
                v = vl + vr + ea16[j] * wev16
                v = jnp.maximum(v, 0.2 * v)
                lg = jnp.sum(v * att16)
                logitv = jnp.where(lane == j, lg, logitv)
                vls.append(vl)
            ex16 = jnp.exp(jnp.clip(logitv, -80.0, 80.0))
            exvb[pl.ds(b0, 16)] = ex16
            for j in range(16):
                xlrb[b0 + j] = ex16[j] * vls[j]
            return 0

        lax.fori_loop(0, B // 16, _group, 0)

    def _scatter(g, b):
        pltpu.sync_copy(xlr[b], num_sh.at[dstv[b]], add=True)
        pltpu.sync_copy(exv[b], den_sh.at[dstv[b]], add=True)
        if emit_ex:
            e0 = ebase + g * B
            pltpu.sync_copy(exv[b], ex_out.at[pl.ds(e0, B)])

    _start_lin(0, 0)
    _wait_lin(0)
    _start_gather(0)
    _start_lin(1, 1)

    def _iter2b(g2, _):
        for b in (0, 1):
            g = g2 * 2 + b
            nb = 1 - b
            _wait_gather(b)

            @pl.when(g + 1 < NCH)
            def _wg():
                _wait_lin(nb)
                _start_gather(nb)

            _compute(b)
            _scatter(g, b)

            @pl.when(g + 2 < NCH)
            def _sl():
                _start_lin(g + 2, b)
        return 0

    lax.fori_loop(0, NCH // 2, _iter2b, 0)
    plsc.subcore_barrier()

    pltpu.sync_copy(num_sh.at[pl.ds(row0, STRIPE), :],
                    num_out.at[c, pl.ds(row0, STRIPE), :])
    pltpu.sync_copy(den_sh.at[pl.ds(row0, STRIPE)],
                    den_out.at[c, pl.ds(row0, STRIPE)])


def _edge_pass(src, dst, ea, xl, xr, params, E, NPAD, STRIPE, emit_ex):
    PER_W = E // NW
    mesh = plsc.VectorSubcoreMesh(core_axis_name="c", subcore_axis_name="s")
    out_type = [jax.ShapeDtypeStruct((NC, NPAD, 16), F32),
                jax.ShapeDtypeStruct((NC, NPAD), F32)]
    if emit_ex:
        out_type.append(jax.ShapeDtypeStruct((E,), F32))
    scratch = (
        [pltpu.VMEM((B,), I32)] * 4 +
        [pltpu.VMEM((B,), F32)] * 2 +
        [pltpu.VMEM((B, 16), F32)] * 4 +
        [pltpu.VMEM((B,), F32)] * 2 +
        [pltpu.VMEM((32,), F32)] +
        [pltpu.VMEM_SHARED((NPAD, 16), F32),
         pltpu.VMEM_SHARED((NPAD,), F32)] +
        [pltpu.SemaphoreType.DMA] * 6
    )
    body = functools.partial(_edge_body, emit_ex, NPAD, STRIPE, PER_W)
    fn = pl.kernel(body, out_type=tuple(out_type), mesh=mesh,
                   scratch_types=tuple(scratch),
                   compiler_params=pltpu.CompilerParams(
                       needs_layout_passes=False, use_tc_tiling_on_sc=False))
    return fn(src, dst, ea, xl, xr, params)


def _alpha_body(PER_W, NPAD, dst_hbm, ex_hbm, invden_hbm, alpha_out,
                dstv0, dstv1, exv0, exv1, av0, av1, dv,
                semI0, semI1):
    dstv = (dstv0, dstv1)
    exv = (exv0, exv1)
    av = (av0, av1)
    semI = (semI0, semI1)
    c = lax.axis_index("c")
    s = lax.axis_index("s")
    w = s * NC + c
    ebase = w * PER_W
    NCH = PER_W // B

    pltpu.sync_copy(invden_hbm, dv)

    def _start_lin(g, b):
        e0 = ebase + g * B
        pltpu.async_copy(dst_hbm.at[pl.ds(e0, B)], dstv[b], semI[b])
        pltpu.async_copy(ex_hbm.at[pl.ds(e0, B)], exv[b], semI[b])

    def _wait_lin(b):
        pltpu.make_async_copy(dst_hbm.at[pl.ds(0, B)], dstv[b], semI[b]).wait()
        pltpu.make_async_copy(ex_hbm.at[pl.ds(0, B)], exv[b], semI[b]).wait()

    def _compute_out(g, b):
        dstvb = dstv[b]
        exvb = exv[b]
        avb = av[b]

        def _group(t, _):
            b0 = t * 16
            idx16 = dstvb[pl.ds(b0, 16)]
            dv16 = plsc.load_gather(dv, [idx16])
            avb[pl.ds(b0, 16)] = exvb[pl.ds(b0, 16)] * dv16
            return 0

        lax.fori_loop(0, B // 16, _group, 0)
        e0 = ebase + g * B
        pltpu.sync_copy(avb, alpha_out.at[pl.ds(e0, B)])

    _start_lin(0, 0)

    def _iter2(g2, _):
        for b in (0, 1):
            g = g2 * 2 + b
            nb = 1 - b
            _wait_lin(b)

            @pl.when(g + 1 < NCH)
            def _sl():
                _start_lin(g + 1, nb)

            _compute_out(g, b)
        return 0

    lax.fori_loop(0, NCH // 2, _iter2, 0)


def _alpha_pass(dst, ex, invden, E, NPAD):
    PER_W = E // NW
    mesh = plsc.VectorSubcoreMesh(core_axis_name="c", subcore_axis_name="s")
    scratch = (
        [pltpu.VMEM((B,), I32)] * 2 +
        [pltpu.VMEM((B,), F32)] * 4 +
        [pltpu.VMEM((NPAD,), F32)] +
        [pltpu.SemaphoreType.DMA] * 2
    )
    fn = pl.kernel(functools.partial(_alpha_body, PER_W, NPAD),
                   out_type=jax.ShapeDtypeStruct((E,), F32),
                   mesh=mesh, scratch_types=tuple(scratch),
                   compiler_params=pltpu.CompilerParams(
                       needs_layout_passes=False, use_tc_tiling_on_sc=False))
    return fn(dst, ex, invden)


def kernel(x, edge_index, edge_attr, Wl1, Wr1, We1, att1, b1, Ws, bs,
           gamma, beta, Wl2, Wr2, We2, att2, b2, Wc1, bc1, Wt1, bt1,
           Wc2, bc2, Wt2, bt2):
    N = x.shape[0]
    E = edge_index.shape[1]
    STRIPE = (((N + NS - 1) // NS) + 399) // 400 * 400
    NPAD = NS * STRIPE
    BN = 512
    assert NPAD % BN == 0

    nch = (E + NW * B - 1) // (NW * B)
    nch += nch % 2
    E_PAD = NW * B * nch

    src = jnp.pad(edge_index[0], (0, E_PAD - E), constant_values=N)
    dst = jnp.pad(edge_index[1], (0, E_PAD - E), constant_values=N)
    ea = jnp.pad(edge_attr[:, 0], (0, E_PAD - E))
    xpad = jnp.pad(x, ((0, NPAD - N), (0, 0)))

    mea = _ea_mean(ea, E)

    xl1, xr1, numi1, deni1 = _tc_node_call(
        _node1_body, 2, [16, 1], NPAD, BN,
        (xpad, mea, Wl1.T, Wr1.T, We1.reshape(1, 16), att1.reshape(1, 16)),
        [pl.BlockSpec((BN, 2), lambda i: (i, 0)), _full((1, 1)),
         _full((2, 16)), _full((2, 16)), _full((1, 16)), _full((1, 16))],
    )

    params1 = jnp.concatenate([We1[:, 0], att1]).astype(F32)
    np1, dp1 = _edge_pass(src, dst, ea, xl1, xr1, params1,
                          E_PAD, NPAD, STRIPE, emit_ex=False)

    bias1 = (b1 + bs).reshape(1, 16)
    geff = (gamma / jnp.sqrt(1.0 + 1e-5)).reshape(1, 16)
    xl2, xr2, num2i, den2i, exs = _tc_node_call(
        _node2_body, 2, [16, 1, 1], NPAD, BN,
        (np1, dp1.reshape(NC, NPAD, 1), numi1, deni1, xpad, mea,
         Ws.T, bias1, geff, beta.reshape(1, 16),
         Wl2.T, Wr2.T, We2.reshape(1, 16), att2.reshape(1, 16)),
        [pl.BlockSpec((NC, BN, 16), lambda i: (0, i, 0)),
         pl.BlockSpec((NC, BN, 1), lambda i: (0, i, 0)),
         pl.BlockSpec((BN, 16), lambda i: (i, 0)),
         pl.BlockSpec((BN, 1), lambda i: (i, 0)),
         pl.BlockSpec((BN, 2), lambda i: (i, 0)), _full((1, 1)),
         _full((2, 16)), _full((1, 16)), _full((1, 16)), _full((1, 16)),
         _full((16, 16)), _full((16, 16)), _full((1, 16)), _full((1, 16))],
    )

    params2 = jnp.concatenate([We2[:, 0], att2]).astype(F32)
    np2, dp2, ex2 = _edge_pass(src, dst, ea, xl2, xr2, params2,
                               E_PAD, NPAD, STRIPE, emit_ex=True)

    hc, ht, invden, aself = _tc_node_call(
        _node3_body, 0, [9, 4, 1, 1], NPAD, BN,
        (np2, dp2.reshape(NC, NPAD, 1), num2i, den2i, exs,
         b2.reshape(1, 16),
         Wc1.T, bc1.reshape(1, 16), Wc2.T, bc2.reshape(1, 9),
         Wt1.T, bt1.reshape(1, 16), Wt2.T, bt2.reshape(1, 4)),
        [pl.BlockSpec((NC, BN, 16), lambda i: (0, i, 0)),
         pl.BlockSpec((NC, BN, 1), lambda i: (0, i, 0)),
         pl.BlockSpec((BN, 16), lambda i: (i, 0)),
         pl.BlockSpec((BN, 1), lambda i: (i, 0)),
         pl.BlockSpec((BN, 1), lambda i: (i, 0)),
         _full((1, 16)),
         _full((16, 16)), _full((1, 16)), _full((16, 9)), _full((1, 9)),
         _full((16, 16)), _full((1, 16)), _full((16, 4)), _full((1, 4))],
    )

    alpha_e = _alpha_pass(dst, ex2, invden[:, 0], E_PAD, NPAD)

    out13 = jnp.concatenate([hc[:N], ht[:N]], axis=1)
    alpha = jnp.concatenate([alpha_e[:E], aself[:N, 0]])
    return (out13, alpha)


# speedup vs baseline: 50.3356x; 1.0120x over previous
import functools

import jax
import jax.numpy as jnp
from jax import lax
from jax.experimental import pallas as pl
from jax.experimental.pallas import tpu as pltpu
from jax.experimental.pallas import tpu_sc as plsc

F32 = jnp.float32
I32 = jnp.int32

L = 16
NC = 2
NS = 16
NW = NC * NS
B = 304


def _mean_body(ea_ref, out_ref):
    i = pl.program_id(0)

    @pl.when(i == 0)
    def _():
        out_ref[...] = jnp.zeros_like(out_ref)

    out_ref[...] += jnp.sum(ea_ref[...]).reshape(1, 1)


def _ea_mean(ea, E):
    rows = ea.shape[0] // 128
    ea2 = ea.reshape(rows, 128)
    grid = next(g for g in (8, 5, 4, 2, 1)
                if rows % g == 0 and (rows // g) % 8 == 0)
    s = pl.pallas_call(
        _mean_body,
        grid=(grid,),
        in_specs=[pl.BlockSpec((rows // grid, 128), lambda i: (i, 0))],
        out_specs=pl.BlockSpec((1, 1), lambda i: (0, 0)),
        out_shape=jax.ShapeDtypeStruct((1, 1), F32),
    )(ea2)
    return s / E


def _lrelu(m):
    return jnp.maximum(m, 0.2 * m)


def _self_attn(xl, xr, mea, wev, attr):
    m = _lrelu(xl + xr + mea * wev)
    logit = jnp.sum(m * attr, axis=1, keepdims=True)
    return jnp.exp(jnp.clip(logit, -80.0, 80.0))


def _node1_body(x_ref, wlt_ref, wrt_ref, xl_ref, xr_ref):
    x = x_ref[...]
    wlt = wlt_ref[...]
    wrt = wrt_ref[...]
    xl_ref[...] = x[:, 0:1] * wlt[0:1, :] + x[:, 1:2] * wlt[1:2, :]
    xr_ref[...] = x[:, 0:1] * wrt[0:1, :] + x[:, 1:2] * wrt[1:2, :]


def _node2_body(np_ref, dp_ref, xl1_ref, xr1_ref, x_ref, mea_ref,
                wev1_ref, att1r_ref,
                wst_ref, bias1_ref, geff_ref, beta_ref,
                wl2t_ref, wr2t_ref,
                xl2_ref, xr2_ref):
    xl1 = xl1_ref[...]
    mea = mea_ref[0, 0]
    ex = _self_attn(xl1, xr1_ref[...], mea, wev1_ref[...], att1r_ref[...])
    num = np_ref[0] + np_ref[1] + ex * xl1
    den = dp_ref[0] + dp_ref[1] + ex
    out1 = num / (den + 1e-16)
    x = x_ref[...]
    wst = wst_ref[...]
    h = out1 + bias1_ref[...] + x[:, 0:1] * wst[0:1, :] + x[:, 1:2] * wst[1:2, :]
    h = geff_ref[...] * h + beta_ref[...]
    h = jnp.where(h > 0, h, jnp.exp(jnp.minimum(h, 0.0)) - 1.0)
    xl2_ref[...] = jnp.dot(h, wl2t_ref[...], preferred_element_type=F32)
    xr2_ref[...] = jnp.dot(h, wr2t_ref[...], preferred_element_type=F32)


def _node3_body(np_ref, dp_ref, xl2_ref, xr2_ref, mea_ref,
                wev2_ref, att2r_ref, b2r_ref,
                wc1t_ref, bc1r_ref, wc2t_ref, bc2r_ref,
                wt1t_ref, bt1r_ref, wt2t_ref, bt2r_ref,
                hc_ref, ht_ref, invden_ref, aself_ref):
    xl2 = xl2_ref[...]
    ex = _self_attn(xl2, xr2_ref[...], mea_ref[0, 0], wev2_ref[...],
                    att2r_ref[...])
    num = np_ref[0] + np_ref[1] + ex * xl2
    den = dp_ref[0] + dp_ref[1] + ex
    invden = 1.0 / (den + 1e-16)
    h = num * invden + b2r_ref[...]
    h = jnp.where(h > 0, h, jnp.exp(jnp.minimum(h, 0.0)) - 1.0)
    hc = jnp.dot(h, wc1t_ref[...], preferred_element_type=F32) + bc1r_ref[...]
    hc = jnp.dot(hc, wc2t_ref[...], preferred_element_type=F32) + bc2r_ref[...]
    ht = jnp.dot(h, wt1t_ref[...], preferred_element_type=F32) + bt1r_ref[...]
    ht = jnp.dot(ht, wt2t_ref[...], preferred_element_type=F32) + bt2r_ref[...]
    hc_ref[...] = hc
    ht_ref[...] = ht
    invden_ref[...] = invden
    aself_ref[...] = ex * invden


def _full(shape):
    nd = len(shape)
    return pl.BlockSpec(shape, lambda i: (0,) * nd)


def _tc_node_call(body, n_out16, extra_outs, NPAD, BN, args, specs):
    grid = NPAD // BN
    outs = [jax.ShapeDtypeStruct((NPAD, 16), F32)] * n_out16
    out_specs = [pl.BlockSpec((BN, 16), lambda i: (i, 0))] * n_out16
    for w in extra_outs:
        outs.append(jax.ShapeDtypeStruct((NPAD, w), F32))
        out_specs.append(pl.BlockSpec((BN, w), lambda i: (i, 0)))
    return pl.pallas_call(
        body,
        grid=(grid,),
        in_specs=specs,
        out_specs=out_specs,
        out_shape=outs,
    )(*args)


def _edge_body(emit_ex, NPAD, STRIPE, PER_W,
               src_hbm, dst_hbm, ea_hbm, xl_hbm, xr_hbm, par_hbm,
               *refs):
    if emit_ex:
        num_out, den_out, ex_out = refs[:3]
        rest = refs[3:]
    else:
        num_out, den_out = refs[:2]
        rest = refs[2:]
    (srcv0, srcv1, dstv0, dstv1, eav0, eav1, xlr0, xlr1, xrr0, xrr1,
     exv0, exv1, parv, num_sh, den_sh, semI0, semI1, semL0, semL1,
     semR0, semR1, semS0, semS1) = rest
    srcv = (srcv0, srcv1)
    dstv = (dstv0, dstv1)
    eav = (eav0, eav1)
    xlr = (xlr0, xlr1)
    xrr = (xrr0, xrr1)
    exv = (exv0, exv1)
    semI = (semI0, semI1)
    semL = (semL0, semL1)
    semR = (semR0, semR1)

    c = lax.axis_index("c")
    s = lax.axis_index("s")
    w = s * NC + c
    row0 = s * STRIPE

    pltpu.sync_copy(par_hbm, parv)
    wev16 = parv[pl.ds(0, 16)]
    att16 = parv[pl.ds(16, 16)]

    def _zrow(i, _):
        xlr0[i] = jnp.zeros((16,), F32)
        return 0

    lax.fori_loop(0, B, _zrow, 0)

    def _zden(i, _):
        eav0[pl.ds(i * 16, 16)] = jnp.zeros((16,), F32)
        return 0

    lax.fori_loop(0, B // 16, _zden, 0)

    off = 0
    while off < STRIPE:
        sz = min(B, STRIPE - off)
        pltpu.sync_copy(xlr0.at[pl.ds(0, sz), :],
                        num_sh.at[pl.ds(row0 + off, sz), :])
        pltpu.sync_copy(eav0.at[pl.ds(0, sz)],
                        den_sh.at[pl.ds(row0 + off, sz)])
        off += sz
    plsc.subcore_barrier()

    ebase = w * PER_W
    NCH = PER_W // B

    def _start_lin(g, b):
        e0 = ebase + g * B
        pltpu.async_copy(src_hbm.at[pl.ds(e0, B)], srcv[b], semI[b])
        pltpu.async_copy(dst_hbm.at[pl.ds(e0, B)], dstv[b], semI[b])
        pltpu.async_copy(ea_hbm.at[pl.ds(e0, B)], eav[b], semI[b])

    def _wait_lin(b):
        pltpu.make_async_copy(src_hbm.at[pl.ds(0, B)], srcv[b], semI[b]).wait()
        pltpu.make_async_copy(dst_hbm.at[pl.ds(0, B)], dstv[b], semI[b]).wait()
        pltpu.make_async_copy(ea_hbm.at[pl.ds(0, B)], eav[b], semI[b]).wait()

    def _start_gather(b):
        pltpu.async_copy(xl_hbm.at[srcv[b]], xlr[b], semL[b])
        pltpu.async_copy(xr_hbm.at[dstv[b]], xrr[b], semR[b])

    def _wait_gather(b):
        pltpu.make_async_copy(xl_hbm.at[srcv[b]], xlr[b], semL[b]).wait()
        pltpu.make_async_copy(xr_hbm.at[dstv[b]], xrr[b], semR[b]).wait()

    def _compute(b):
        xlrb = xlr[b]
        xrrb = xrr[b]
        eavb = eav[b]
        exvb = exv[b]

        def _group(t, _):
            b0 = t * 16
            ea16 = eavb[pl.ds(b0, 16)]
            lane = lax.iota(I32, 16)
            logitv = jnp.zeros((16,), F32)
            vls = []
            for j in range(16):
                vl = xlrb[b0 + j]
                vr = xrrb[b0 + j]
                v = vl + vr + ea16[j] * wev16
                v = jnp.maximum(v, 0.2 * v)
                lg = jnp.sum(v * att16)
                logitv = jnp.where(lane == j, lg, logitv)
                vls.append(vl)
            ex16 = jnp.exp(jnp.clip(logitv, -80.0, 80.0))
            exvb[pl.ds(b0, 16)] = ex16
            for j in range(16):
                xlrb[b0 + j] = ex16[j] * vls[j]
            return 0

        lax.fori_loop(0, B // 16, _group, 0)

    def _scatter(g, b):
        pltpu.sync_copy(xlr[b], num_sh.at[dstv[b]], add=True)
        pltpu.sync_copy(exv[b], den_sh.at[dstv[b]], add=True)
        if emit_ex:
            e0 = ebase + g * B
            pltpu.sync_copy(exv[b], ex_out.at[pl.ds(e0, B)])

    _start_lin(0, 0)
    _wait_lin(0)
    _start_gather(0)
    _start_lin(1, 1)

    def _iter2b(g2, _):
        for b in (0, 1):
            g = g2 * 2 + b
            nb = 1 - b
            _wait_gather(b)

            @pl.when(g + 1 < NCH)
            def _wg():
                _wait_lin(nb)
                _start_gather(nb)

            _compute(b)
            _scatter(g, b)

            @pl.when(g + 2 < NCH)
            def _sl():
                _start_lin(g + 2, b)
        return 0

    lax.fori_loop(0, NCH // 2, _iter2b, 0)
    plsc.subcore_barrier()

    pltpu.sync_copy(num_sh.at[pl.ds(row0, STRIPE), :],
                    num_out.at[c, pl.ds(row0, STRIPE), :])
    pltpu.sync_copy(den_sh.at[pl.ds(row0, STRIPE)],
                    den_out.at[c, pl.ds(row0, STRIPE)])


def _edge_pass(src, dst, ea, xl, xr, params, E, NPAD, STRIPE, emit_ex):
    PER_W = E // NW
    mesh = plsc.VectorSubcoreMesh(core_axis_name="c", subcore_axis_name="s")
    out_type = [jax.ShapeDtypeStruct((NC, NPAD, 16), F32),
                jax.ShapeDtypeStruct((NC, NPAD), F32)]
    if emit_ex:
        out_type.append(jax.ShapeDtypeStruct((E,), F32))
    scratch = (
        [pltpu.VMEM((B,), I32)] * 4 +
        [pltpu.VMEM((B,), F32)] * 2 +
        [pltpu.VMEM((B, 16), F32)] * 4 +
        [pltpu.VMEM((B,), F32)] * 2 +
        [pltpu.VMEM((32,), F32)] +
        [pltpu.VMEM_SHARED((NPAD, 16), F32),
         pltpu.VMEM_SHARED((NPAD,), F32)] +
        [pltpu.SemaphoreType.DMA] * 8
    )
    body = functools.partial(_edge_body, emit_ex, NPAD, STRIPE, PER_W)
    fn = pl.kernel(body, out_type=tuple(out_type), mesh=mesh,
                   scratch_types=tuple(scratch),
                   compiler_params=pltpu.CompilerParams(
                       needs_layout_passes=False, use_tc_tiling_on_sc=False))
    return fn(src, dst, ea, xl, xr, params)


def _alpha_body(PER_W, NPAD, dst_hbm, ex_hbm, invden_hbm, alpha_out,
                dstv0, dstv1, exv0, exv1, av0, av1, dv,
                semI0, semI1):
    dstv = (dstv0, dstv1)
    exv = (exv0, exv1)
    av = (av0, av1)
    semI = (semI0, semI1)
    c = lax.axis_index("c")
    s = lax.axis_index("s")
    w = s * NC + c
    ebase = w * PER_W
    NCH = PER_W // B

    pltpu.sync_copy(invden_hbm, dv)

    def _start_lin(g, b):
        e0 = ebase + g * B
        pltpu.async_copy(dst_hbm.at[pl.ds(e0, B)], dstv[b], semI[b])
        pltpu.async_copy(ex_hbm.at[pl.ds(e0, B)], exv[b], semI[b])

    def _wait_lin(b):
        pltpu.make_async_copy(dst_hbm.at[pl.ds(0, B)], dstv[b], semI[b]).wait()
        pltpu.make_async_copy(ex_hbm.at[pl.ds(0, B)], exv[b], semI[b]).wait()

    def _compute_out(g, b):
        dstvb = dstv[b]
        exvb = exv[b]
        avb = av[b]

        def _group(t, _):
            b0 = t * 16
            idx16 = dstvb[pl.ds(b0, 16)]
            dv16 = plsc.load_gather(dv, [idx16])
            avb[pl.ds(b0, 16)] = exvb[pl.ds(b0, 16)] * dv16
            return 0

        lax.fori_loop(0, B // 16, _group, 0)
        e0 = ebase + g * B
        pltpu.sync_copy(avb, alpha_out.at[pl.ds(e0, B)])

    _start_lin(0, 0)

    def _iter2(g2, _):
        for b in (0, 1):
            g = g2 * 2 + b
            nb = 1 - b
            _wait_lin(b)

            @pl.when(g + 1 < NCH)
            def _sl():
                _start_lin(g + 1, nb)

            _compute_out(g, b)
        return 0

    lax.fori_loop(0, NCH // 2, _iter2, 0)


def _alpha_pass(dst, ex, invden, E, NPAD):
    PER_W = E // NW
    mesh = plsc.VectorSubcoreMesh(core_axis_name="c", subcore_axis_name="s")
    scratch = (
        [pltpu.VMEM((B,), I32)] * 2 +
        [pltpu.VMEM((B,), F32)] * 4 +
        [pltpu.VMEM((NPAD,), F32)] +
        [pltpu.SemaphoreType.DMA] * 2
    )
    fn = pl.kernel(functools.partial(_alpha_body, PER_W, NPAD),
                   out_type=jax.ShapeDtypeStruct((E,), F32),
                   mesh=mesh, scratch_types=tuple(scratch),
                   compiler_params=pltpu.CompilerParams(
                       needs_layout_passes=False, use_tc_tiling_on_sc=False))
    return fn(dst, ex, invden)


def kernel(x, edge_index, edge_attr, Wl1, Wr1, We1, att1, b1, Ws, bs,
           gamma, beta, Wl2, Wr2, We2, att2, b2, Wc1, bc1, Wt1, bt1,
           Wc2, bc2, Wt2, bt2):
    N = x.shape[0]
    E = edge_index.shape[1]
    STRIPE = (((N + NS - 1) // NS) + 399) // 400 * 400
    NPAD = NS * STRIPE
    BN = 512
    assert NPAD % BN == 0

    nch = (E + NW * B - 1) // (NW * B)
    nch += nch % 2
    E_PAD = NW * B * nch

    src = jnp.pad(edge_index[0], (0, E_PAD - E), constant_values=N)
    dst = jnp.pad(edge_index[1], (0, E_PAD - E), constant_values=N)
    ea = jnp.pad(edge_attr[:, 0], (0, E_PAD - E))
    xpad = jnp.pad(x, ((0, NPAD - N), (0, 0)))

    mea = _ea_mean(ea, E)

    xl1, xr1 = _tc_node_call(
        _node1_body, 2, [], NPAD, BN,
        (xpad, Wl1.T, Wr1.T),
        [pl.BlockSpec((BN, 2), lambda i: (i, 0)),
         _full((2, 16)), _full((2, 16))],
    )

    params1 = jnp.concatenate([We1[:, 0], att1]).astype(F32)
    np1, dp1 = _edge_pass(src, dst, ea, xl1, xr1, params1,
                          E_PAD, NPAD, STRIPE, emit_ex=False)

    bias1 = (b1 + bs).reshape(1, 16)
    geff = (gamma / jnp.sqrt(1.0 + 1e-5)).reshape(1, 16)
    xl2, xr2 = _tc_node_call(
        _node2_body, 2, [], NPAD, BN,
        (np1, dp1.reshape(NC, NPAD, 1), xl1, xr1, xpad, mea,
         We1.reshape(1, 16), att1.reshape(1, 16),
         Ws.T, bias1, geff, beta.reshape(1, 16),
         Wl2.T, Wr2.T),
        [pl.BlockSpec((NC, BN, 16), lambda i: (0, i, 0)),
         pl.BlockSpec((NC, BN, 1), lambda i: (0, i, 0)),
         pl.BlockSpec((BN, 16), lambda i: (i, 0)),
         pl.BlockSpec((BN, 16), lambda i: (i, 0)),
         pl.BlockSpec((BN, 2), lambda i: (i, 0)), _full((1, 1)),
         _full((1, 16)), _full((1, 16)),
         _full((2, 16)), _full((1, 16)), _full((1, 16)), _full((1, 16)),
         _full((16, 16)), _full((16, 16))],
    )

    params2 = jnp.concatenate([We2[:, 0], att2]).astype(F32)
    np2, dp2, ex2 = _edge_pass(src, dst, ea, xl2, xr2, params2,
                               E_PAD, NPAD, STRIPE, emit_ex=True)

    hc, ht, invden, aself = _tc_node_call(
        _node3_body, 0, [9, 4, 1, 1], NPAD, BN,
        (np2, dp2.reshape(NC, NPAD, 1), xl2, xr2, mea,
         We2.reshape(1, 16), att2.reshape(1, 16), b2.reshape(1, 16),
         Wc1.T, bc1.reshape(1, 16), Wc2.T, bc2.reshape(1, 9),
         Wt1.T, bt1.reshape(1, 16), Wt2.T, bt2.reshape(1, 4)),
        [pl.BlockSpec((NC, BN, 16), lambda i: (0, i, 0)),
         pl.BlockSpec((NC, BN, 1), lambda i: (0, i, 0)),
         pl.BlockSpec((BN, 16), lambda i: (i, 0)),
         pl.BlockSpec((BN, 16), lambda i: (i, 0)), _full((1, 1)),
         _full((1, 16)), _full((1, 16)), _full((1, 16)),
         _full((16, 16)), _full((1, 16)), _full((16, 9)), _full((1, 9)),
         _full((16, 16)), _full((1, 16)), _full((16, 4)), _full((1, 4))],
    )

    alpha_e = _alpha_pass(dst, ex2, invden[:, 0], E_PAD, NPAD)

    out13 = jnp.concatenate([hc[:N], ht[:N]], axis=1)
    alpha = jnp.concatenate([alpha_e[:E], aself[:N, 0]])
    return (out13, alpha)
